# baseline (speedup 1.0000x reference)
import functools

import jax
import jax.numpy as jnp
from jax import lax
from jax.experimental import pallas as pl
from jax.experimental.pallas import tpu as pltpu
from jax.experimental.pallas import tpu_sc as plsc

B = 16384
D = 64
L = 16
NC = 2
NS = 16
NW = NC * NS
BPW = B // NW
NSEG = 4
SEG = BPW // NSEG
NG = BPW // L

ROWS = 1000000
SUB = 8
GRAN = 16


def _gmf_body(idx_u_hbm, idx_i_hbm, ut_hbm, it_hbm, w_hbm, b_hbm, out_hbm,
              idx_u_v, idx_i_v, u4a_v, i4a_v, u4b_v, i4b_v, w_v, wspl_v,
              b_v, out_v, sem_a, sem_b):
    wid = lax.axis_index("s") * NC + lax.axis_index("c")
    base = wid * BPW

    pltpu.sync_copy(idx_u_hbm.at[pl.ds(wid * NSEG, NSEG)], idx_u_v)
    pltpu.sync_copy(idx_i_hbm.at[pl.ds(wid * NSEG, NSEG)], idx_i_v)
    pltpu.sync_copy(w_hbm, w_v)
    pltpu.sync_copy(b_hbm, b_v)

    for d in range(D):
        wspl_v[pl.ds(d * L, L)] = plsc.load_gather(
            w_v, [jnp.full((L,), d, jnp.int32)])

    bvec = b_v[...]
    iota = lax.iota(jnp.int32, L)
    lane_masks = [iota == r for r in range(L)]
    zero = jnp.zeros((L,), jnp.int32)
    iota16 = iota * GRAN

    def load_ivs(g):
        j = g >> 3
        co = (g & 7) * L
        return idx_u_v[j, pl.ds(co, L)], idx_i_v[j, pl.ds(co, L)]

    def issue(g, u4_v, i4_v, sem):
        iv_u, iv_i = load_ivs(g)
        for r in range(L):
            su = jnp.max(jnp.where(lane_masks[r], iv_u, zero))
            si = jnp.max(jnp.where(lane_masks[r], iv_i, zero))
            ou = pl.multiple_of(su & -16, 128)
            oi = pl.multiple_of(si & -16, 128)
            pltpu.async_copy(
                ut_hbm.at[:, :, pl.ds(ou, GRAN)],
                u4_v.at[:, :, pl.ds(r * GRAN, GRAN)], sem)
            pltpu.async_copy(
                it_hbm.at[:, :, pl.ds(oi, GRAN)],
                i4_v.at[:, :, pl.ds(r * GRAN, GRAN)], sem)

    def drain(u4_v, i4_v, sem):
        for r in range(L):
            pltpu.make_async_copy(
                ut_hbm.at[:, :, pl.ds(0, GRAN)],
                u4_v.at[:, :, pl.ds(r * GRAN, GRAN)], sem).wait()
            pltpu.make_async_copy(
                it_hbm.at[:, :, pl.ds(0, GRAN)],
                i4_v.at[:, :, pl.ds(r * GRAN, GRAN)], sem).wait()

    def compute(g, u4_v, i4_v):
        iv_u, iv_i = load_ivs(g)
        cols_u = iota16 + (iv_u & 15)
        cols_i = iota16 + (iv_i & 15)
        acc = jnp.zeros((L,), jnp.float32)
        for d in range(D):
            kk = jnp.full((L,), d // SUB, jnp.int32)
            ss = jnp.full((L,), d % SUB, jnp.int32)
            u = plsc.load_gather(u4_v, [kk, ss, cols_u])
            i = plsc.load_gather(i4_v, [kk, ss, cols_i])
            acc = acc + (u * i) * wspl_v[pl.ds(d * L, L)]
        x = acc + bvec
        off = pl.multiple_of(g * L, L)
        out_v[pl.ds(off, L)] = 1.0 / (1.0 + jnp.exp(-x))

    issue(0, u4a_v, i4a_v, sem_a)

    def pipe(t, carry):
        g0 = t * 2
        issue(g0 + 1, u4b_v, i4b_v, sem_b)
        drain(u4a_v, i4a_v, sem_a)
        compute(g0, u4a_v, i4a_v)

        @pl.when(t != NG // 2 - 1)
        def _():
            issue(g0 + 2, u4a_v, i4a_v, sem_a)

        drain(u4b_v, i4b_v, sem_b)
        compute(g0 + 1, u4b_v, i4b_v)
        return carry

    lax.fori_loop(0, NG // 2, pipe, 0)

    pltpu.sync_copy(out_v, out_hbm.at[pl.ds(base, BPW)])


_gmf_call = functools.partial(
    pl.kernel,
    mesh=plsc.VectorSubcoreMesh(core_axis_name="c", subcore_axis_name="s"),
    out_type=jax.ShapeDtypeStruct((B,), jnp.float32),
    compiler_params=pltpu.CompilerParams(
        needs_layout_passes=False, use_tc_tiling_on_sc=True),
    scratch_types=[
        pltpu.VMEM((NSEG, SEG), jnp.int32),
        pltpu.VMEM((NSEG, SEG), jnp.int32),
        pltpu.VMEM((SUB, SUB, L * GRAN), jnp.float32),
        pltpu.VMEM((SUB, SUB, L * GRAN), jnp.float32),
        pltpu.VMEM((SUB, SUB, L * GRAN), jnp.float32),
        pltpu.VMEM((SUB, SUB, L * GRAN), jnp.float32),
        pltpu.VMEM((D,), jnp.float32),
        pltpu.VMEM((D * L,), jnp.float32),
        pltpu.VMEM((L,), jnp.float32),
        pltpu.VMEM((BPW,), jnp.float32),
        pltpu.SemaphoreType.DMA,
        pltpu.SemaphoreType.DMA,
    ],
)


def kernel(user_entries, item_entries, user_table, item_table, W, b):
    idx_u = user_entries.astype(jnp.int32).reshape(NW * NSEG, SEG)
    idx_i = item_entries.astype(jnp.int32).reshape(NW * NSEG, SEG)
    ut3 = user_table.T.reshape(SUB, SUB, ROWS)
    it3 = item_table.T.reshape(SUB, SUB, ROWS)
    w_flat = W.astype(jnp.float32).reshape(D)
    b16 = jnp.broadcast_to(b.astype(jnp.float32).reshape(()), (L,))
    return _gmf_call(_gmf_body)(idx_u, idx_i, ut3, it3, w_flat, b16)


# speedup vs baseline: 1.0289x; 1.0289x over previous
import functools

import jax
import jax.numpy as jnp
from jax import lax
from jax.experimental import pallas as pl
from jax.experimental.pallas import tpu as pltpu
from jax.experimental.pallas import tpu_sc as plsc

B = 16384
D = 64
L = 16
NC = 2
NS = 16
NW = NC * NS
BPW = B // NW
NSEG = 4
SEG = BPW // NSEG
NG = BPW // L

ROWS = 1000000
SUB = 8
GRAN = 16


def _gmf_body(idx_u_hbm, idx_i_hbm, ut_hbm, it_hbm, w_hbm, b_hbm, out_hbm,
              idx_u_v, idx_i_v, u4a_v, i4a_v, u4b_v, i4b_v, w_v, wspl_v,
              b_v, out_v, sem_a, sem_b):
    wid = lax.axis_index("s") * NC + lax.axis_index("c")
    base = wid * BPW

    pltpu.sync_copy(idx_u_hbm.at[pl.ds(wid * NSEG, NSEG)], idx_u_v)
    pltpu.sync_copy(idx_i_hbm.at[pl.ds(wid * NSEG, NSEG)], idx_i_v)
    pltpu.sync_copy(w_hbm, w_v)
    pltpu.sync_copy(b_hbm, b_v)

    for d in range(D):
        wspl_v[pl.ds(d * L, L)] = plsc.load_gather(
            w_v, [jnp.full((L,), d, jnp.int32)])

    bvec = b_v[...]
    iota = lax.iota(jnp.int32, L)
    lane_masks = [iota == r for r in range(L)]
    zero = jnp.zeros((L,), jnp.int32)
    iota16 = iota * GRAN

    def load_ivs(g):
        j = g >> 3
        co = (g & 7) * L
        return idx_u_v[j, pl.ds(co, L)], idx_i_v[j, pl.ds(co, L)]

    def issue(g, u4_v, i4_v, sem):
        iv_u, iv_i = load_ivs(g)
        for r in range(L):
            su = iv_u[r]
            si = iv_i[r]
            ou = pl.multiple_of(su & -16, 128)
            oi = pl.multiple_of(si & -16, 128)
            pltpu.async_copy(
                ut_hbm.at[:, :, pl.ds(ou, GRAN)],
                u4_v.at[:, :, pl.ds(r * GRAN, GRAN)], sem)
            pltpu.async_copy(
                it_hbm.at[:, :, pl.ds(oi, GRAN)],
                i4_v.at[:, :, pl.ds(r * GRAN, GRAN)], sem)

    def drain(u4_v, i4_v, sem):
        for r in range(L):
            pltpu.make_async_copy(
                ut_hbm.at[:, :, pl.ds(0, GRAN)],
                u4_v.at[:, :, pl.ds(r * GRAN, GRAN)], sem).wait()
            pltpu.make_async_copy(
                it_hbm.at[:, :, pl.ds(0, GRAN)],
                i4_v.at[:, :, pl.ds(r * GRAN, GRAN)], sem).wait()

    def compute(g, u4_v, i4_v):
        iv_u, iv_i = load_ivs(g)
        cols_u = iota16 + (iv_u & 15)
        cols_i = iota16 + (iv_i & 15)
        acc = jnp.zeros((L,), jnp.float32)
        for d in range(D):
            kk = jnp.full((L,), d // SUB, jnp.int32)
            ss = jnp.full((L,), d % SUB, jnp.int32)
            u = plsc.load_gather(u4_v, [kk, ss, cols_u])
            i = plsc.load_gather(i4_v, [kk, ss, cols_i])
            acc = acc + (u * i) * wspl_v[pl.ds(d * L, L)]
        x = acc + bvec
        off = pl.multiple_of(g * L, L)
        out_v[pl.ds(off, L)] = 1.0 / (1.0 + jnp.exp(-x))

    issue(0, u4a_v, i4a_v, sem_a)

    def pipe(t, carry):
        g0 = t * 2
        issue(g0 + 1, u4b_v, i4b_v, sem_b)
        drain(u4a_v, i4a_v, sem_a)
        compute(g0, u4a_v, i4a_v)

        @pl.when(t != NG // 2 - 1)
        def _():
            issue(g0 + 2, u4a_v, i4a_v, sem_a)

        drain(u4b_v, i4b_v, sem_b)
        compute(g0 + 1, u4b_v, i4b_v)
        return carry

    lax.fori_loop(0, NG // 2, pipe, 0)

    pltpu.sync_copy(out_v, out_hbm.at[pl.ds(base, BPW)])


_gmf_call = functools.partial(
    pl.kernel,
    mesh=plsc.VectorSubcoreMesh(core_axis_name="c", subcore_axis_name="s"),
    out_type=jax.ShapeDtypeStruct((B,), jnp.float32),
    compiler_params=pltpu.CompilerParams(
        needs_layout_passes=False, use_tc_tiling_on_sc=True),
    scratch_types=[
        pltpu.VMEM((NSEG, SEG), jnp.int32),
        pltpu.VMEM((NSEG, SEG), jnp.int32),
        pltpu.VMEM((SUB, SUB, L * GRAN), jnp.float32),
        pltpu.VMEM((SUB, SUB, L * GRAN), jnp.float32),
        pltpu.VMEM((SUB, SUB, L * GRAN), jnp.float32),
        pltpu.VMEM((SUB, SUB, L * GRAN), jnp.float32),
        pltpu.VMEM((D,), jnp.float32),
        pltpu.VMEM((D * L,), jnp.float32),
        pltpu.VMEM((L,), jnp.float32),
        pltpu.VMEM((BPW,), jnp.float32),
        pltpu.SemaphoreType.DMA,
        pltpu.SemaphoreType.DMA,
    ],
)


def kernel(user_entries, item_entries, user_table, item_table, W, b):
    idx_u = user_entries.astype(jnp.int32).reshape(NW * NSEG, SEG)
    idx_i = item_entries.astype(jnp.int32).reshape(NW * NSEG, SEG)
    ut3 = user_table.T.reshape(SUB, SUB, ROWS)
    it3 = item_table.T.reshape(SUB, SUB, ROWS)
    w_flat = W.astype(jnp.float32).reshape(D)
    b16 = jnp.broadcast_to(b.astype(jnp.float32).reshape(()), (L,))
    return _gmf_call(_gmf_body)(idx_u, idx_i, ut3, it3, w_flat, b16)


# speedup vs baseline: 1.0437x; 1.0144x over previous
import functools

import jax
import jax.numpy as jnp
from jax import lax
from jax.experimental import pallas as pl
from jax.experimental.pallas import tpu as pltpu
from jax.experimental.pallas import tpu_sc as plsc

B = 16384
D = 64
L = 16
NC = 2
NS = 16
NW = NC * NS
BPW = B // NW
NSEG = 4
SEG = BPW // NSEG
NG = BPW // L

ROWS = 1000000
SUB = 8
GRAN = 16


def _gmf_body(idx_u_hbm, idx_i_hbm, ut_hbm, it_hbm, w_hbm, b_hbm, out_hbm,
              idx_u_v, idx_i_v, u4a_v, i4a_v, u4b_v, i4b_v, w_v, wspl_v,
              b_v, out_v, sem_a, sem_b):
    wid = lax.axis_index("s") * NC + lax.axis_index("c")
    base = wid * BPW

    pltpu.sync_copy(idx_u_hbm.at[pl.ds(wid * NSEG, NSEG)], idx_u_v)
    pltpu.sync_copy(idx_i_hbm.at[pl.ds(wid * NSEG, NSEG)], idx_i_v)
    pltpu.sync_copy(w_hbm, w_v)
    pltpu.sync_copy(b_hbm, b_v)

    for d in range(D):
        wspl_v[pl.ds(d * L, L)] = plsc.load_gather(
            w_v, [jnp.full((L,), d, jnp.int32)])

    bvec = b_v[...]
    iota16 = lax.iota(jnp.int32, L) * GRAN

    def load_ivs(g):
        j = g >> 3
        co = (g & 7) * L
        return idx_u_v[j, pl.ds(co, L)], idx_i_v[j, pl.ds(co, L)]

    def issue(g, u4_v, i4_v, sem):
        iv_u, iv_i = load_ivs(g)
        for r in range(L):
            su = iv_u[r]
            si = iv_i[r]
            ou = pl.multiple_of(su & -16, 128)
            oi = pl.multiple_of(si & -16, 128)
            pltpu.async_copy(
                ut_hbm.at[:, :, pl.ds(ou, GRAN)],
                u4_v.at[:, :, pl.ds(r * GRAN, GRAN)], sem)
            pltpu.async_copy(
                it_hbm.at[:, :, pl.ds(oi, GRAN)],
                i4_v.at[:, :, pl.ds(r * GRAN, GRAN)], sem)

    def drain(u4_v, i4_v, sem):
        for r in range(L):
            pltpu.make_async_copy(
                ut_hbm.at[:, :, pl.ds(0, GRAN)],
                u4_v.at[:, :, pl.ds(r * GRAN, GRAN)], sem).wait()
            pltpu.make_async_copy(
                it_hbm.at[:, :, pl.ds(0, GRAN)],
                i4_v.at[:, :, pl.ds(r * GRAN, GRAN)], sem).wait()

    def compute(g, u4_v, i4_v):
        iv_u, iv_i = load_ivs(g)
        cols_u = iota16 + (iv_u & 15)
        cols_i = iota16 + (iv_i & 15)
        acc = jnp.zeros((L,), jnp.float32)
        for d in range(D):
            kk = jnp.full((L,), d // SUB, jnp.int32)
            ss = jnp.full((L,), d % SUB, jnp.int32)
            u = plsc.load_gather(u4_v, [kk, ss, cols_u])
            i = plsc.load_gather(i4_v, [kk, ss, cols_i])
            acc = acc + (u * i) * wspl_v[pl.ds(d * L, L)]
        x = acc + bvec
        off = pl.multiple_of(g * L, L)
        out_v[pl.ds(off, L)] = 1.0 / (1.0 + jnp.exp(-x))

    issue(0, u4a_v, i4a_v, sem_a)

    def pipe(t, carry):
        g0 = t * 2
        issue(g0 + 1, u4b_v, i4b_v, sem_b)
        drain(u4a_v, i4a_v, sem_a)
        compute(g0, u4a_v, i4a_v)

        @pl.when(t != NG // 2 - 1)
        def _():
            issue(g0 + 2, u4a_v, i4a_v, sem_a)

        drain(u4b_v, i4b_v, sem_b)
        compute(g0 + 1, u4b_v, i4b_v)
        return carry

    lax.fori_loop(0, NG // 2, pipe, 0)

    pltpu.sync_copy(out_v, out_hbm.at[pl.ds(base, BPW)])


_gmf_call = functools.partial(
    pl.kernel,
    mesh=plsc.VectorSubcoreMesh(core_axis_name="c", subcore_axis_name="s"),
    out_type=jax.ShapeDtypeStruct((B,), jnp.float32),
    compiler_params=pltpu.CompilerParams(
        needs_layout_passes=False, use_tc_tiling_on_sc=True),
    scratch_types=[
        pltpu.VMEM((NSEG, SEG), jnp.int32),
        pltpu.VMEM((NSEG, SEG), jnp.int32),
        pltpu.VMEM((SUB, SUB, L * GRAN), jnp.float32),
        pltpu.VMEM((SUB, SUB, L * GRAN), jnp.float32),
        pltpu.VMEM((SUB, SUB, L * GRAN), jnp.float32),
        pltpu.VMEM((SUB, SUB, L * GRAN), jnp.float32),
        pltpu.VMEM((D,), jnp.float32),
        pltpu.VMEM((D * L,), jnp.float32),
        pltpu.VMEM((L,), jnp.float32),
        pltpu.VMEM((BPW,), jnp.float32),
        pltpu.SemaphoreType.DMA,
        pltpu.SemaphoreType.DMA,
    ],
)


def kernel(user_entries, item_entries, user_table, item_table, W, b):
    idx_u = user_entries.astype(jnp.int32).reshape(NW * NSEG, SEG)
    idx_i = item_entries.astype(jnp.int32).reshape(NW * NSEG, SEG)
    ut3 = user_table.T.reshape(SUB, SUB, ROWS)
    it3 = item_table.T.reshape(SUB, SUB, ROWS)
    w_flat = W.astype(jnp.float32).reshape(D)
    b16 = jnp.broadcast_to(b.astype(jnp.float32).reshape(()), (L,))
    return _gmf_call(_gmf_body)(idx_u, idx_i, ut3, it3, w_flat, b16)
